# baseline (speedup 1.0000x reference)
import functools

import jax
import jax.numpy as jnp
from jax import lax
from jax.experimental import pallas as pl
from jax.experimental.pallas import tpu as pltpu
from jax.experimental.pallas import tpu_sc as plsc

B = 64
N_SAMPLES = 64000
F = 512
STRIDE = 256
N_FRAMES = (N_SAMPLES - F) // STRIDE + 1
N_CHUNKS = N_SAMPLES // STRIDE

NUM_CORES = 2
NUM_SUBCORES = 16
NUM_WORKERS = NUM_CORES * NUM_SUBCORES
ROWS_PER_WORKER = B // NUM_WORKERS

_mesh = plsc.VectorSubcoreMesh(core_axis_name="c", subcore_axis_name="s")


GROUPS = (25,) * 9 + (24,)
GMAX = max(GROUPS)


@functools.partial(
    pl.kernel,
    mesh=_mesh,
    out_type=jax.ShapeDtypeStruct((B, 1, N_FRAMES, F), jnp.float32),
    scratch_types=[
        pltpu.VMEM((GMAX + 1, STRIDE), jnp.float32),
        pltpu.VMEM((GMAX + 1, STRIDE), jnp.float32),
        pltpu.SemaphoreType.DMA,
        pltpu.SemaphoreType.DMA,
        pltpu.SemaphoreType.DMA,
        pltpu.SemaphoreType.DMA,
    ],
    compiler_params=pltpu.CompilerParams(use_tc_tiling_on_sc=False),
)
def _frames_kernel(sig_hbm, out_hbm, buf0, buf1, sem_i0, sem_i1, sem_o0, sem_o1):
    wid = lax.axis_index("s") * NUM_CORES + lax.axis_index("c")
    bufs = (buf0, buf1)
    in_sems = (sem_i0, sem_i1)
    out_sems = (sem_o0, sem_o1)
    pending = [(), ()]
    gi = 0
    for r in range(ROWS_PER_WORKER):
        b = wid * ROWS_PER_WORKER + r
        f0 = 0
        for n in GROUPS:
            k = gi % 2
            gi += 1
            buf = bufs[k]
            for cp in pending[k]:
                cp.wait()
            load = pltpu.async_copy(
                sig_hbm.at[b, pl.ds(f0, n + 1)],
                buf.at[pl.ds(0, n + 1)],
                in_sems[k],
            )
            load.wait()
            pending[k] = (
                pltpu.async_copy(
                    buf.at[pl.ds(0, n)],
                    out_hbm.at[b, 0, pl.ds(f0, n), pl.ds(0, STRIDE)],
                    out_sems[k],
                ),
                pltpu.async_copy(
                    buf.at[pl.ds(1, n)],
                    out_hbm.at[b, 0, pl.ds(f0, n), pl.ds(STRIDE, STRIDE)],
                    out_sems[k],
                ),
            )
            f0 += n
    for cps in pending:
        for cp in cps:
            cp.wait()


def kernel(sig):
    return _frames_kernel(sig.reshape(B, N_CHUNKS, STRIDE))


# speedup vs baseline: 1.3002x; 1.3002x over previous
import functools

import jax
import jax.numpy as jnp
from jax import lax
from jax.experimental import pallas as pl
from jax.experimental.pallas import tpu as pltpu
from jax.experimental.pallas import tpu_sc as plsc

B = 64
N_SAMPLES = 64000
F = 512
STRIDE = 256
N_FRAMES = (N_SAMPLES - F) // STRIDE + 1
N_CHUNKS = N_SAMPLES // STRIDE

NUM_CORES = 2
NUM_SUBCORES = 16
NUM_WORKERS = NUM_CORES * NUM_SUBCORES
ROWS_PER_WORKER = B // NUM_WORKERS

_mesh = plsc.VectorSubcoreMesh(core_axis_name="c", subcore_axis_name="s")


@functools.partial(
    pl.kernel,
    mesh=_mesh,
    out_type=jax.ShapeDtypeStruct((B, 1, N_FRAMES, F), jnp.float32),
    scratch_types=[
        pltpu.VMEM((N_CHUNKS, STRIDE), jnp.float32),
        pltpu.VMEM((N_CHUNKS, STRIDE), jnp.float32),
        pltpu.SemaphoreType.DMA,
        pltpu.SemaphoreType.DMA,
        pltpu.SemaphoreType.DMA,
    ],
    compiler_params=pltpu.CompilerParams(use_tc_tiling_on_sc=False),
)
def _frames_kernel(sig_hbm, out_hbm, buf0, buf1, sem_in, sem_o0, sem_o1):
    wid = lax.axis_index("s") * NUM_CORES + lax.axis_index("c")
    bufs = (buf0, buf1)
    out_sems = (sem_o0, sem_o1)
    stores = []
    for r in range(ROWS_PER_WORKER):
        b = wid * ROWS_PER_WORKER + r
        buf = bufs[r % 2]
        pltpu.async_copy(sig_hbm.at[b], buf, sem_in).wait()
        stores.append(
            pltpu.async_copy(
                buf.at[pl.ds(0, N_FRAMES)],
                out_hbm.at[b, 0, :, pl.ds(0, STRIDE)],
                out_sems[r % 2],
            )
        )
        stores.append(
            pltpu.async_copy(
                buf.at[pl.ds(1, N_FRAMES)],
                out_hbm.at[b, 0, :, pl.ds(STRIDE, STRIDE)],
                out_sems[r % 2],
            )
        )
    for cp in stores:
        cp.wait()


def kernel(sig):
    return _frames_kernel(sig.reshape(B, N_CHUNKS, STRIDE))
